# baseline (speedup 1.0000x reference)
import jax
import jax.numpy as jnp
from jax.experimental import pallas as pl
from jax.experimental.pallas import tpu as pltpu

_MU = 0.02
_BINS = 10
_ALPHA = 0.75
_N = 8388608

_COLS = 1024
_ROWS = _N // _COLS
_BLK = 512
_GRID = _ROWS // _BLK
_CH = 8
_NACC = 21


def _red(v):
    return jnp.sum(v.reshape(_CH, 8, 128), axis=1)


def _body(p_ref, t_ref, out_ref, acc_ref):
    step = pl.program_id(0)

    @pl.when(step == 0)
    def _init():
        acc_ref[...] = jnp.zeros_like(acc_ref)

    def chunk2(i, accs):
        p = p_ref[pl.ds(i * _CH, _CH), :]
        t = t_ref[pl.ds(i * _CH, _CH), :]
        d = jnp.abs(p - t)
        loss = jnp.where(d < _MU, (0.5 / _MU) * d * d, d - 0.5 * _MU)
        m = jnp.abs(jnp.tanh(p) - jnp.tanh(t)) * float(_BINS)
        new = list(accs)
        new[0] = accs[0] + _red(loss)
        for k in range(1, _BINS):
            mask = m >= float(k)
            new[k] = accs[k] + _red(jnp.where(mask, loss, 0.0))
            new[_BINS + k] = accs[_BINS + k] + _red(
                jnp.where(mask, 1.0, 0.0))
        new[2 * _BINS] = accs[2 * _BINS] + _red(
            jnp.where(m >= float(_BINS), 1.0, 0.0))
        return tuple(new)

    zero = jnp.zeros((_CH, 128), jnp.float32)
    init = tuple(zero for _ in range(_NACC))
    accs = jax.lax.fori_loop(0, _BLK // _CH, chunk2, init)
    for k in range(_NACC):
        if k == _BINS:
            continue
        acc_ref[k, :, :] += accs[k]

    @pl.when(step == _GRID - 1)
    def _finish():
        ls_ge = [jnp.sum(acc_ref[k]) for k in range(_BINS)]
        cnt_ge = [jnp.float32(_N)] + [
            jnp.sum(acc_ref[_BINS + k]) for k in range(1, _BINS + 1)]
        lanes = jax.lax.broadcasted_iota(jnp.int32, (8, 128), 1) + \
            128 * jax.lax.broadcasted_iota(jnp.int32, (8, 128), 0)
        tot_v = jnp.ones((8, 128), jnp.float32)
        ls_v = jnp.zeros((8, 128), jnp.float32)
        for b in range(_BINS):
            cnt_b = cnt_ge[b] - cnt_ge[b + 1]
            ls_b = ls_ge[b] - (ls_ge[b + 1] if b + 1 < _BINS else 0.0)
            tot_v = jnp.where(lanes == b, jnp.maximum(cnt_b, 1.0), tot_v)
            ls_v = jnp.where(lanes == b, ls_b, ls_v)
        w_v = jnp.exp(-_ALPHA * jnp.log(tot_v))
        out_ref[0, 0] = jnp.sum(ls_v * w_v) * (1.0 / _N)


def kernel(pred, target):
    p2 = pred.reshape(_ROWS, _COLS)
    t2 = target.reshape(_ROWS, _COLS)
    out = pl.pallas_call(
        _body,
        grid=(_GRID,),
        in_specs=[
            pl.BlockSpec((_BLK, _COLS), lambda i: (i, 0)),
            pl.BlockSpec((_BLK, _COLS), lambda i: (i, 0)),
        ],
        out_specs=pl.BlockSpec(memory_space=pltpu.SMEM),
        out_shape=jax.ShapeDtypeStruct((1, 1), jnp.float32),
        scratch_shapes=[pltpu.VMEM((_NACC + 1, 8, 128), jnp.float32)],
        compiler_params=pltpu.CompilerParams(
            dimension_semantics=("arbitrary",)),
    )(p2, t2)
    return out[0, 0]


# speedup vs baseline: 2.8770x; 2.8770x over previous
import jax
import jax.numpy as jnp
from jax.experimental import pallas as pl
from jax.experimental.pallas import tpu as pltpu

_MU = 0.02
_BINS = 10
_ALPHA = 0.75
_N = 8388608

_COLS = 1024
_ROWS = _N // _COLS
_BLK = 512
_GRID = _ROWS // _BLK
_CH = 8
_NACC = 21


def _red(v):
    acc = v[:, 0:128]
    for j in range(1, _COLS // 128):
        acc = acc + v[:, j * 128:(j + 1) * 128]
    return acc


def _body(p_ref, t_ref, out_ref, acc_ref):
    step = pl.program_id(0)

    @pl.when(step == 0)
    def _init():
        acc_ref[...] = jnp.zeros_like(acc_ref)

    def chunk2(i, accs):
        p = p_ref[pl.ds(i * _CH, _CH), :]
        t = t_ref[pl.ds(i * _CH, _CH), :]
        d = jnp.abs(p - t)
        loss = jnp.where(d < _MU, (0.5 / _MU) * d * d, d - 0.5 * _MU)
        m = jnp.abs(jnp.tanh(p) - jnp.tanh(t)) * float(_BINS)
        new = list(accs)
        new[0] = accs[0] + _red(loss)
        for k in range(1, _BINS):
            mask = m >= float(k)
            new[k] = accs[k] + _red(jnp.where(mask, loss, 0.0))
            new[_BINS + k] = accs[_BINS + k] + _red(
                jnp.where(mask, 1.0, 0.0))
        new[2 * _BINS] = accs[2 * _BINS] + _red(
            jnp.where(m >= float(_BINS), 1.0, 0.0))
        return tuple(new)

    zero = jnp.zeros((_CH, 128), jnp.float32)
    init = tuple(zero for _ in range(_NACC))
    accs = jax.lax.fori_loop(0, _BLK // _CH, chunk2, init)
    for k in range(_NACC):
        if k == _BINS:
            continue
        acc_ref[k, :, :] += accs[k]

    @pl.when(step == _GRID - 1)
    def _finish():
        ls_ge = [jnp.sum(acc_ref[k]) for k in range(_BINS)]
        cnt_ge = [jnp.float32(_N)] + [
            jnp.sum(acc_ref[_BINS + k]) for k in range(1, _BINS + 1)]
        lanes = jax.lax.broadcasted_iota(jnp.int32, (8, 128), 1) + \
            128 * jax.lax.broadcasted_iota(jnp.int32, (8, 128), 0)
        tot_v = jnp.ones((8, 128), jnp.float32)
        ls_v = jnp.zeros((8, 128), jnp.float32)
        for b in range(_BINS):
            cnt_b = cnt_ge[b] - cnt_ge[b + 1]
            ls_b = ls_ge[b] - (ls_ge[b + 1] if b + 1 < _BINS else 0.0)
            tot_v = jnp.where(lanes == b, jnp.maximum(cnt_b, 1.0), tot_v)
            ls_v = jnp.where(lanes == b, ls_b, ls_v)
        w_v = jnp.exp(-_ALPHA * jnp.log(tot_v))
        out_ref[0, 0] = jnp.sum(ls_v * w_v) * (1.0 / _N)


def kernel(pred, target):
    p2 = pred.reshape(_ROWS, _COLS)
    t2 = target.reshape(_ROWS, _COLS)
    out = pl.pallas_call(
        _body,
        grid=(_GRID,),
        in_specs=[
            pl.BlockSpec((_BLK, _COLS), lambda i: (i, 0)),
            pl.BlockSpec((_BLK, _COLS), lambda i: (i, 0)),
        ],
        out_specs=pl.BlockSpec(memory_space=pltpu.SMEM),
        out_shape=jax.ShapeDtypeStruct((1, 1), jnp.float32),
        scratch_shapes=[pltpu.VMEM((_NACC + 1, 8, 128), jnp.float32)],
        compiler_params=pltpu.CompilerParams(
            dimension_semantics=("arbitrary",)),
    )(p2, t2)
    return out[0, 0]
